# baseline (speedup 1.0000x reference)
import dataclasses
import functools

import jax
import jax.numpy as jnp
from jax.experimental import pallas as pl
from jax.experimental.pallas import tpu as pltpu
from jax.experimental.pallas import tpu_sc as plsc

N_PAD = 1792
N_WORKERS = 32
ROWS_PER = N_PAD // N_WORKERS
EDGE_CHUNK = 1728


def _build_adjacency_sc(src_p, dst_p, w_p):
    num_edges = src_p.shape[0]

    cp = pltpu.CompilerParams()
    if "needs_layout_passes" in pltpu.CompilerParams.__dataclass_fields__:
        cp = dataclasses.replace(cp, needs_layout_passes=False)

    @functools.partial(
        pl.kernel,
        out_type=jax.ShapeDtypeStruct((N_PAD, N_PAD), jnp.float32),
        mesh=plsc.VectorSubcoreMesh(core_axis_name="c", subcore_axis_name="s"),
        scratch_types=[
            pltpu.VMEM((ROWS_PER, N_PAD), jnp.float32),
            pltpu.VMEM((EDGE_CHUNK,), jnp.int32),
            pltpu.VMEM((EDGE_CHUNK,), jnp.int32),
            pltpu.VMEM((EDGE_CHUNK,), jnp.float32),
            pltpu.SemaphoreType.DMA,
        ],
        compiler_params=cp,
    )
    def build(src_hbm, dst_hbm, w_hbm, z_hbm, m_hbm, mband, sbuf, dbuf, wbuf,
              sem):
        cid = jax.lax.axis_index("c")
        sid = jax.lax.axis_index("s")
        lo = (cid * 16 + sid) * ROWS_PER

        pltpu.async_copy(z_hbm, mband, sem).wait()

        @pl.loop(0, num_edges, step=EDGE_CHUNK)
        def _(e0):
            pltpu.async_copy(src_hbm.at[pl.ds(e0, EDGE_CHUNK)], sbuf, sem).wait()
            pltpu.async_copy(dst_hbm.at[pl.ds(e0, EDGE_CHUNK)], dbuf, sem).wait()
            pltpu.async_copy(w_hbm.at[pl.ds(e0, EDGE_CHUNK)], wbuf, sem).wait()

            @pl.loop(0, EDGE_CHUNK, step=16)
            def _(i):
                s = sbuf[pl.ds(i, 16)]
                d = dbuf[pl.ds(i, 16)]
                w = wbuf[pl.ds(i, 16)]
                msk = (s >= lo) & (s < lo + ROWS_PER)
                r = jnp.clip(s - lo, 0, ROWS_PER - 1)
                plsc.addupdate_scatter(mband, [r, d], w, mask=msk)

        pltpu.sync_copy(mband, m_hbm.at[pl.ds(lo, ROWS_PER)])

    zeros_band = jnp.zeros((ROWS_PER, N_PAD), jnp.float32)
    return build(src_p, dst_p, w_p, zeros_band)


def _group_pool_matrix(c):
    ri = jax.lax.broadcasted_iota(jnp.int32, (c, c), 0) // 8
    ci = jax.lax.broadcasted_iota(jnp.int32, (c, c), 1) // 8
    return (ri == ci).astype(jnp.float32)


def _f32dot(a, b):
    return jax.lax.dot(a, b, preferred_element_type=jnp.float32)


def _gn_relu(x, g, b, mask, n_valid):
    c = x.shape[0]
    pool = _group_pool_matrix(c)
    cnt = 8.0 * n_valid
    s = jnp.sum(x, axis=1, keepdims=True)
    mean = _f32dot(pool, s) / cnt
    xm = (x - mean) * mask
    s2 = jnp.sum(xm * xm, axis=1, keepdims=True)
    var = _f32dot(pool, s2) / cnt
    a = g * jax.lax.rsqrt(var + 1e-5)
    y = xm * a + b * mask
    return jnp.maximum(y, 0.0)


def _linear(w, b, x, mask):
    return _f32dot(w, x) + b * mask


def _res_block(x, w, mask, n_valid, m, has_skip):
    y = _gn_relu(x, w["pre_g"], w["pre_b"], mask, n_valid)
    y = _linear(w["lin1_W"], w["lin1_b"], y, mask)
    y = _gn_relu(y, w["n1_g"], w["n1_b"], mask, n_valid)
    sup = _f32dot(w["conv_W"], y)
    agg = jax.lax.dot(sup.astype(jnp.bfloat16), m,
                      preferred_element_type=jnp.float32) + w["conv_b"] * mask
    y = _gn_relu(agg, w["n2_g"], w["n2_b"], mask, n_valid)
    y = _linear(w["lin2_W"], w["lin2_b"], y, mask)
    if has_skip:
        x = _linear(w["skip_W"], w["skip_b"], x, mask)
    return x + y


def _forward_body(block_metas, n_valid, *refs):
    it = iter(refs)
    pc_ref = next(it)
    m_ref = next(it)
    refv_ref = next(it)

    def take(n):
        return [next(it)[...] for _ in range(n)]

    w1, b1, w2, b2, w_ref3, w_img, gc_b = take(7)
    blocks = []
    for has_skip in block_metas:
        names = ["pre_g", "pre_b", "lin1_W", "lin1_b", "n1_g", "n1_b",
                 "conv_W", "conv_b", "n2_g", "n2_b", "lin2_W", "lin2_b"]
        if has_skip:
            names += ["skip_W", "skip_b"]
        blocks.append(dict(zip(names, take(len(names)))))
    sn_g, sn_b, sl_w, sl_b = take(4)
    out_ref = next(it)

    mask = (jax.lax.broadcasted_iota(jnp.int32, (1, N_PAD), 1)
            < n_valid).astype(jnp.float32)
    m = m_ref[...]

    pc = pc_ref[0]
    h = jnp.maximum(_f32dot(w1, pc) + b1, 0.0)
    h = jnp.maximum(_f32dot(w2, h) + b2, 0.0)
    feat = jnp.max(h, axis=1, keepdims=True)

    x = (_f32dot(w_ref3, refv_ref[...]) + _f32dot(w_img, feat) + gc_b) * mask

    for has_skip, w in zip(block_metas, blocks):
        x = _res_block(x, w, mask, n_valid, m, has_skip)

    x = _gn_relu(x, sn_g, sn_b, mask, n_valid)
    out_ref[0] = _f32dot(sl_w, x) + sl_b * mask


def kernel(sparse_pc, batch_size, params, src, dst, edge_w):
    del batch_size
    b = sparse_pc.shape[0]
    n = params["ref_vertices"].shape[1]
    e = src.shape[0]

    e_pad = -(-e // EDGE_CHUNK) * EDGE_CHUNK
    src_p = jnp.pad(src, (0, e_pad - e))
    dst_p = jnp.pad(dst, (0, e_pad - e))
    w_p = jnp.pad(edge_w, (0, e_pad - e))
    m = _build_adjacency_sc(src_p, dst_p, w_p).astype(jnp.bfloat16)

    def col(v):
        return v.reshape(-1, 1)

    refv = jnp.pad(params["ref_vertices"], ((0, 0), (0, N_PAD - n)))
    ops = [
        params["res_w1"], col(params["res_b1"]),
        params["res_w2"], col(params["res_b2"]),
        params["gc_lin_W"][:, :3], params["gc_lin_W"][:, 3:],
        col(params["gc_lin_b"]),
    ]
    block_metas = []
    for bp in list(params["blocks"]) + list(params["shape_blocks"]):
        has_skip = "skip_W" in bp
        block_metas.append(has_skip)
        ops += [col(bp["pre_g"]), col(bp["pre_b"]),
                bp["lin1_W"], col(bp["lin1_b"]),
                col(bp["n1_g"]), col(bp["n1_b"]),
                bp["conv_W"].T, col(bp["conv_b"]),
                col(bp["n2_g"]), col(bp["n2_b"]),
                bp["lin2_W"], col(bp["lin2_b"])]
        if has_skip:
            ops += [bp["skip_W"], col(bp["skip_b"])]
    ops += [col(params["shape_norm_g"]), col(params["shape_norm_b"]),
            params["shape_lin_W"], col(params["shape_lin_b"])]

    def const_spec(arr):
        nd = arr.ndim
        return pl.BlockSpec(arr.shape, lambda i, _n=nd: (0,) * _n)

    in_specs = (
        [pl.BlockSpec((1, 3, sparse_pc.shape[2]), lambda i: (i, 0, 0)),
         pl.BlockSpec((N_PAD, N_PAD), lambda i: (0, 0)),
         const_spec(refv)]
        + [const_spec(a) for a in ops]
    )

    out = pl.pallas_call(
        functools.partial(_forward_body, tuple(block_metas), n),
        grid=(b,),
        in_specs=in_specs,
        out_specs=pl.BlockSpec((1, 3, N_PAD), lambda i: (i, 0, 0)),
        out_shape=jax.ShapeDtypeStruct((b, 3, N_PAD), jnp.float32),
        compiler_params=pltpu.CompilerParams(
            dimension_semantics=("parallel",)),
    )(sparse_pc, m, refv, *ops)

    return out[:, :, :n]


# speedup vs baseline: 1.2736x; 1.2736x over previous
import dataclasses
import functools

import jax
import jax.numpy as jnp
from jax.experimental import pallas as pl
from jax.experimental.pallas import tpu as pltpu
from jax.experimental.pallas import tpu_sc as plsc

N_PAD = 1792
N_WORKERS = 32
ROWS_PER = N_PAD // N_WORKERS
EDGE_CHUNK = 1728


def _build_adjacency_sc(src_p, dst_p, w_p):
    num_edges = src_p.shape[0]

    cp = pltpu.CompilerParams()
    if "needs_layout_passes" in pltpu.CompilerParams.__dataclass_fields__:
        cp = dataclasses.replace(cp, needs_layout_passes=False)

    @functools.partial(
        pl.kernel,
        out_type=jax.ShapeDtypeStruct((N_PAD, N_PAD), jnp.float32),
        mesh=plsc.VectorSubcoreMesh(core_axis_name="c", subcore_axis_name="s"),
        scratch_types=[
            pltpu.VMEM((ROWS_PER, N_PAD), jnp.float32),
            pltpu.VMEM((EDGE_CHUNK,), jnp.int32),
            pltpu.VMEM((EDGE_CHUNK,), jnp.int32),
            pltpu.VMEM((EDGE_CHUNK,), jnp.float32),
            pltpu.SemaphoreType.DMA,
        ],
        compiler_params=cp,
    )
    def build(src_hbm, dst_hbm, w_hbm, z_hbm, m_hbm, mband, sbuf, dbuf, wbuf,
              sem):
        cid = jax.lax.axis_index("c")
        sid = jax.lax.axis_index("s")
        lo = (cid * 16 + sid) * ROWS_PER

        pltpu.async_copy(z_hbm, mband, sem).wait()

        @pl.loop(0, num_edges, step=EDGE_CHUNK)
        def _(e0):
            pltpu.async_copy(src_hbm.at[pl.ds(e0, EDGE_CHUNK)], sbuf, sem).wait()
            pltpu.async_copy(dst_hbm.at[pl.ds(e0, EDGE_CHUNK)], dbuf, sem).wait()
            pltpu.async_copy(w_hbm.at[pl.ds(e0, EDGE_CHUNK)], wbuf, sem).wait()

            @pl.loop(0, EDGE_CHUNK, step=16)
            def _(i):
                s = sbuf[pl.ds(i, 16)]
                d = dbuf[pl.ds(i, 16)]
                w = wbuf[pl.ds(i, 16)]
                msk = (s >= lo) & (s < lo + ROWS_PER)
                r = jnp.clip(s - lo, 0, ROWS_PER - 1)
                plsc.addupdate_scatter(mband, [r, d], w, mask=msk)

        pltpu.sync_copy(mband, m_hbm.at[pl.ds(lo, ROWS_PER)])

    zeros_band = jnp.zeros((ROWS_PER, N_PAD), jnp.float32)
    return build(src_p, dst_p, w_p, zeros_band)


def _group_pool_matrix(c):
    ri = jax.lax.broadcasted_iota(jnp.int32, (c, c), 0) // 8
    ci = jax.lax.broadcasted_iota(jnp.int32, (c, c), 1) // 8
    return (ri == ci).astype(jnp.float32)


def _f32dot(a, b):
    return jax.lax.dot(a, b, preferred_element_type=jnp.float32)


def _gn_relu(x, g, b, n_valid):
    c = x.shape[0]
    pool = _group_pool_matrix(c)
    cnt = 8.0 * n_valid
    xv = x[:, :n_valid]
    s = jnp.sum(xv, axis=1, keepdims=True)
    s2 = jnp.sum(xv * xv, axis=1, keepdims=True)
    mean = _f32dot(pool, s) / cnt
    var = _f32dot(pool, s2) / cnt - mean * mean
    a = g * jax.lax.rsqrt(var + 1e-5)
    return jnp.maximum(x * a + (b - mean * a), 0.0)


def _linear(w, b, x):
    return _f32dot(w, x) + b


def _res_block(x, w, n_valid, m, has_skip):
    y = _gn_relu(x, w["pre_g"], w["pre_b"], n_valid)
    y = _linear(w["lin1_W"], w["lin1_b"], y)
    y = _gn_relu(y, w["n1_g"], w["n1_b"], n_valid)
    sup = _f32dot(w["conv_W"], y)
    agg = jax.lax.dot(sup.astype(jnp.bfloat16), m,
                      preferred_element_type=jnp.float32) + w["conv_b"]
    y = _gn_relu(agg, w["n2_g"], w["n2_b"], n_valid)
    y = _linear(w["lin2_W"], w["lin2_b"], y)
    if has_skip:
        x = _linear(w["skip_W"], w["skip_b"], x)
    return x + y


def _forward_body(block_metas, n_valid, *refs):
    it = iter(refs)
    pc_ref = next(it)
    m_ref = next(it)
    refv_ref = next(it)

    def take(n):
        return [next(it)[...] for _ in range(n)]

    w1, b1, w2, b2, w_ref3, w_img, gc_b = take(7)
    blocks = []
    for has_skip in block_metas:
        names = ["pre_g", "pre_b", "lin1_W", "lin1_b", "n1_g", "n1_b",
                 "conv_W", "conv_b", "n2_g", "n2_b", "lin2_W", "lin2_b"]
        if has_skip:
            names += ["skip_W", "skip_b"]
        blocks.append(dict(zip(names, take(len(names)))))
    sn_g, sn_b, sl_w, sl_b = take(4)
    out_ref = next(it)

    m = m_ref[...]

    pc = pc_ref[0]
    h = jnp.maximum(_f32dot(w1, pc) + b1, 0.0)
    h = jnp.maximum(_f32dot(w2, h) + b2, 0.0)
    feat = jnp.max(h, axis=1, keepdims=True)

    x = _f32dot(w_ref3, refv_ref[...]) + _f32dot(w_img, feat) + gc_b

    for has_skip, w in zip(block_metas, blocks):
        x = _res_block(x, w, n_valid, m, has_skip)

    x = _gn_relu(x, sn_g, sn_b, n_valid)
    out_ref[0] = _f32dot(sl_w, x) + sl_b


def kernel(sparse_pc, batch_size, params, src, dst, edge_w):
    del batch_size
    b = sparse_pc.shape[0]
    n = params["ref_vertices"].shape[1]
    e = src.shape[0]

    e_pad = -(-e // EDGE_CHUNK) * EDGE_CHUNK
    src_p = jnp.pad(src, (0, e_pad - e))
    dst_p = jnp.pad(dst, (0, e_pad - e))
    w_p = jnp.pad(edge_w, (0, e_pad - e))
    m = _build_adjacency_sc(src_p, dst_p, w_p).astype(jnp.bfloat16)

    def col(v):
        return v.reshape(-1, 1)

    refv = jnp.pad(params["ref_vertices"], ((0, 0), (0, N_PAD - n)))
    ops = [
        params["res_w1"], col(params["res_b1"]),
        params["res_w2"], col(params["res_b2"]),
        params["gc_lin_W"][:, :3], params["gc_lin_W"][:, 3:],
        col(params["gc_lin_b"]),
    ]
    block_metas = []
    for bp in list(params["blocks"]) + list(params["shape_blocks"]):
        has_skip = "skip_W" in bp
        block_metas.append(has_skip)
        ops += [col(bp["pre_g"]), col(bp["pre_b"]),
                bp["lin1_W"], col(bp["lin1_b"]),
                col(bp["n1_g"]), col(bp["n1_b"]),
                bp["conv_W"].T, col(bp["conv_b"]),
                col(bp["n2_g"]), col(bp["n2_b"]),
                bp["lin2_W"], col(bp["lin2_b"])]
        if has_skip:
            ops += [bp["skip_W"], col(bp["skip_b"])]
    ops += [col(params["shape_norm_g"]), col(params["shape_norm_b"]),
            params["shape_lin_W"], col(params["shape_lin_b"])]

    def const_spec(arr):
        nd = arr.ndim
        return pl.BlockSpec(arr.shape, lambda i, _n=nd: (0,) * _n)

    in_specs = (
        [pl.BlockSpec((1, 3, sparse_pc.shape[2]), lambda i: (i, 0, 0)),
         pl.BlockSpec((N_PAD, N_PAD), lambda i: (0, 0)),
         const_spec(refv)]
        + [const_spec(a) for a in ops]
    )

    out = pl.pallas_call(
        functools.partial(_forward_body, tuple(block_metas), n),
        grid=(b,),
        in_specs=in_specs,
        out_specs=pl.BlockSpec((1, 3, N_PAD), lambda i: (i, 0, 0)),
        out_shape=jax.ShapeDtypeStruct((b, 3, N_PAD), jnp.float32),
        compiler_params=pltpu.CompilerParams(
            dimension_semantics=("parallel",)),
    )(sparse_pc, m, refv, *ops)

    return out[:, :, :n]
